# Initial kernel scaffold; baseline (speedup 1.0000x reference)
import jax
import jax.numpy as jnp
from jax.experimental import pallas as pl


def kernel(f, neighbors, wq, wk):
    raise NotImplementedError("write your pallas kernel here")


# speedup vs baseline: 16.9788x; 16.9788x over previous
import jax
import jax.numpy as jnp
from jax.experimental import pallas as pl


def _row_block(n):
    for b in (512, 400, 256, 200, 128, 104, 100, 80, 64, 40, 32, 16, 8):
        if n % b == 0:
            return b
    return n


def _dot(q, k):
    return jax.lax.dot_general(
        q, k, (((1,), (0,)), ((), ())),
        preferred_element_type=jnp.float32,
    )


def _sums_kernel(ft_ref, f_ref, wq_ref, wk_ref, nbr_ref, out_ref):
    q = ft_ref[...] * wq_ref[...]
    k = f_ref[...] * wk_ref[...]
    s = jnp.exp(_dot(q, k))
    masked = jnp.where(nbr_ref[...] != 0, s, 0.0)
    out_ref[0, 0, :] = jnp.sum(masked, axis=1)


def _final_kernel(ft_ref, f_ref, wq_ref, wk_ref, sums_ref, out_ref):
    q = ft_ref[...] * wq_ref[...]
    k = f_ref[...] * wk_ref[...]
    s = jnp.exp(_dot(q, k))
    out_ref[...] = s / sums_ref[...]


@jax.jit
def kernel(f, neighbors, wq, wk):
    size, n = f.shape
    b = _row_block(n)
    grid = (n // b,)
    ft = f.T
    wq_r = wq.reshape(1, size)
    wk_c = wk.reshape(size, 1)

    common_specs = [
        pl.BlockSpec((b, size), lambda i: (i, 0)),
        pl.BlockSpec((size, n), lambda i: (0, 0)),
        pl.BlockSpec((1, size), lambda i: (0, 0)),
        pl.BlockSpec((size, 1), lambda i: (0, 0)),
    ]

    sums = pl.pallas_call(
        _sums_kernel,
        grid=grid,
        in_specs=common_specs + [
            pl.BlockSpec((b, n), lambda i: (i, 0)),
        ],
        out_specs=pl.BlockSpec((1, 1, b), lambda i: (i, 0, 0)),
        out_shape=jax.ShapeDtypeStruct((n // b, 1, b), jnp.float32),
    )(ft, f, wq_r, wk_c, neighbors)
    sums = sums.reshape(1, n)

    out = pl.pallas_call(
        _final_kernel,
        grid=grid,
        in_specs=common_specs + [
            pl.BlockSpec((1, n), lambda i: (0, 0)),
        ],
        out_specs=pl.BlockSpec((b, n), lambda i: (i, 0)),
        out_shape=jax.ShapeDtypeStruct((n, n), jnp.float32),
    )(ft, f, wq_r, wk_c, sums)
    return out
